# baseline (speedup 1.0000x reference)
import functools

import jax
import jax.numpy as jnp
from jax import lax
from jax.experimental import pallas as pl
from jax.experimental.pallas import tpu as pltpu
from jax.experimental.pallas import tpu_sc as plsc

NC = 2
NS = 16
L = 16
NW = NC * NS

B = 16384
K = 32
BPW = B // NW
CHUNK = 128
NCHUNK = BPW // CHUNK

_mesh = plsc.VectorSubcoreMesh(
    core_axis_name="c", subcore_axis_name="s", num_cores=NC, num_subcores=NS
)


@functools.partial(
    pl.kernel,
    out_type=jax.ShapeDtypeStruct((B,), jnp.float32),
    mesh=_mesh,
    compiler_params=pltpu.CompilerParams(
        needs_layout_passes=False, use_tc_tiling_on_sc=False),
    scratch_types=[
        pltpu.VMEM((BPW, 2), jnp.int32),
        pltpu.VMEM((BPW,), jnp.int32),
        pltpu.VMEM((BPW,), jnp.int32),
        pltpu.VMEM((BPW, K), jnp.float32),
        pltpu.VMEM((BPW, K), jnp.float32),
        pltpu.VMEM((BPW,), jnp.float32),
        pltpu.SemaphoreType.DMA,
    ],
)
def _mf_fwd(x_hbm, utab_hbm, vtab_hbm, out_hbm,
            x_v, uidx_v, vidx_v, urows_v, vrows_v, out_v, sem):
    wid = lax.axis_index("s") * NC + lax.axis_index("c")
    base = wid * BPW

    pltpu.sync_copy(x_hbm.at[pl.ds(base, BPW)], x_v)

    lane16 = lax.iota(jnp.int32, L)
    col0 = jnp.zeros((L,), jnp.int32)
    col1 = jnp.ones((L,), jnp.int32)
    for g in range(BPW // L):
        rows = g * L + lane16
        uidx_v[pl.ds(g * L, L)] = plsc.load_gather(x_v, [rows, col0])
        vidx_v[pl.ds(g * L, L)] = plsc.load_gather(x_v, [rows, col1])

    copies = []
    for j in range(NCHUNK):
        sl = pl.ds(j * CHUNK, CHUNK)
        copies.append(
            pltpu.async_copy(utab_hbm.at[uidx_v.at[sl]], urows_v.at[sl], sem))
        copies.append(
            pltpu.async_copy(vtab_hbm.at[vidx_v.at[sl]], vrows_v.at[sl], sem))
    for c in copies:
        c.wait()

    lane = lax.iota(jnp.int32, L)

    def g_body(g, carry):
        rows = g * L + lane
        acc = jnp.zeros((L,), jnp.float32)
        for k in range(K):
            kv = jnp.full((L,), k, jnp.int32)
            u = plsc.load_gather(urows_v, [rows, kv])
            v = plsc.load_gather(vrows_v, [rows, kv])
            acc = acc + u * v
        out_v[pl.ds(g * L, L)] = acc
        return carry

    lax.fori_loop(0, BPW // L, g_body, 0)

    pltpu.sync_copy(out_v, out_hbm.at[pl.ds(base, BPW)])


def kernel(x, user_table, item_table):
    return _mf_fwd(x.astype(jnp.int32), user_table, item_table)


# speedup vs baseline: 2.8727x; 2.8727x over previous
import functools

import jax
import jax.numpy as jnp
from jax import lax
from jax.experimental import pallas as pl
from jax.experimental.pallas import tpu as pltpu
from jax.experimental.pallas import tpu_sc as plsc

NC = 2
NS = 16
L = 16
NW = NC * NS

B = 16384
K = 32
BPW = B // NW
NBUF = 8
W = 128

_mesh = plsc.VectorSubcoreMesh(
    core_axis_name="c", subcore_axis_name="s", num_cores=NC, num_subcores=NS
)

_params = pltpu.CompilerParams(needs_layout_passes=False)


def _gather_scratch(extra):
    sc = [
        pltpu.VMEM((BPW + L,), jnp.int32),
        pltpu.VMEM((BPW * K,), jnp.float32),
    ] + extra
    sc += [pltpu.VMEM((K, W), jnp.float32) for _ in range(NBUF)]
    sc += [pltpu.SemaphoreType.DMA for _ in range(NBUF)]
    return sc


def _ring_gather(idx_hbm, tab_hbm, base, idx_v, rows, bufs, sems):
    pltpu.sync_copy(idx_hbm.at[pl.ds(base, BPW)], idx_v.at[pl.ds(0, BPW)])

    lane = lax.iota(jnp.int32, L)

    def fire(pair, s):
        i0 = idx_v[pl.ds(pair, L)][0]
        t = pl.multiple_of((i0 // W) * W, W)
        return pltpu.async_copy(tab_hbm.at[:, pl.ds(t, W)], bufs[s], sems[s])

    def extract(pair, s):
        i0 = idx_v[pl.ds(pair, L)][0]
        j = jnp.full((L,), i0 % W, jnp.int32)
        rows[pl.ds(pair * K, L)] = plsc.load_gather(bufs[s], [lane, j])
        rows[pl.ds(pair * K + L, L)] = plsc.load_gather(bufs[s], [lane + L, j])

    def batch_body(g, carry):
        pair0 = g * NBUF
        copies = [fire(pair0 + s, s) for s in range(NBUF)]
        for s in range(NBUF):
            copies[s].wait()
            extract(pair0 + s, s)
        return carry

    lax.fori_loop(0, BPW // NBUF, batch_body, 0)


@functools.partial(
    pl.kernel,
    out_type=jax.ShapeDtypeStruct((B * K,), jnp.float32),
    mesh=_mesh,
    compiler_params=_params,
    scratch_types=_gather_scratch([]),
)
def _gather_u(uidx_hbm, ut_hbm, urows_hbm, idx_v, rows, *bufs_sems):
    bufs, sems = bufs_sems[:NBUF], bufs_sems[NBUF:]
    wid = lax.axis_index("s") * NC + lax.axis_index("c")
    base = wid * BPW
    _ring_gather(uidx_hbm, ut_hbm, base, idx_v, rows, bufs, sems)
    pltpu.sync_copy(rows, urows_hbm.at[pl.ds(base * K, BPW * K)])


@functools.partial(
    pl.kernel,
    out_type=jax.ShapeDtypeStruct((B,), jnp.float32),
    mesh=_mesh,
    compiler_params=_params,
    scratch_types=_gather_scratch([
        pltpu.VMEM((BPW * K,), jnp.float32),
        pltpu.VMEM((BPW,), jnp.float32),
    ]),
)
def _gather_v_dot(vidx_hbm, vt_hbm, urows_hbm, out_hbm,
                  idx_v, rows, urows, out_v, *bufs_sems):
    bufs, sems = bufs_sems[:NBUF], bufs_sems[NBUF:]
    wid = lax.axis_index("s") * NC + lax.axis_index("c")
    base = wid * BPW

    pltpu.sync_copy(urows_hbm.at[pl.ds(base * K, BPW * K)], urows)
    _ring_gather(vidx_hbm, vt_hbm, base, idx_v, rows, bufs, sems)

    lane = lax.iota(jnp.int32, L)

    def g_body(g, carry):
        flat = (g * L + lane) * K
        acc = jnp.zeros((L,), jnp.float32)
        for k in range(K):
            acc = acc + (plsc.load_gather(urows, [flat + k]) *
                         plsc.load_gather(rows, [flat + k]))
        out_v[pl.ds(g * L, L)] = acc
        return carry

    lax.fori_loop(0, BPW // L, g_body, 0)

    pltpu.sync_copy(out_v, out_hbm.at[pl.ds(base, BPW)])


def kernel(x, user_table, item_table):
    urows = _gather_u(x[:, 0], user_table.T)
    return _gather_v_dot(x[:, 1], item_table.T, urows)


# speedup vs baseline: 3.1350x; 1.0913x over previous
import functools

import jax
import jax.numpy as jnp
from jax import lax
from jax.experimental import pallas as pl
from jax.experimental.pallas import tpu as pltpu
from jax.experimental.pallas import tpu_sc as plsc

NC = 2
NS = 16
L = 16
NW = NC * NS

B = 16384
K = 32
BPW = B // NW
NBUF = 16
W = 128

_mesh = plsc.VectorSubcoreMesh(
    core_axis_name="c", subcore_axis_name="s", num_cores=NC, num_subcores=NS
)

_params = pltpu.CompilerParams(needs_layout_passes=False)


def _gather_scratch(extra):
    sc = [
        pltpu.VMEM((BPW + L,), jnp.int32),
        pltpu.VMEM((BPW * K,), jnp.float32),
    ] + extra
    sc += [pltpu.VMEM((K, W), jnp.float32) for _ in range(NBUF)]
    sc += [pltpu.SemaphoreType.DMA for _ in range(NBUF)]
    return sc


def _ring_gather(idx_hbm, tab_hbm, base, idx_v, rows, bufs, sems):
    pltpu.sync_copy(idx_hbm.at[pl.ds(base, BPW)], idx_v.at[pl.ds(0, BPW)])

    lane = lax.iota(jnp.int32, L)

    def fire(pair, s):
        i0 = idx_v[pl.ds(pair, L)][0]
        t = pl.multiple_of((i0 // W) * W, W)
        return pltpu.async_copy(tab_hbm.at[:, pl.ds(t, W)], bufs[s], sems[s])

    def extract(pair, s):
        i0 = idx_v[pl.ds(pair, L)][0]
        j = jnp.full((L,), i0 % W, jnp.int32)
        rows[pl.ds(pair * K, L)] = plsc.load_gather(bufs[s], [lane, j])
        rows[pl.ds(pair * K + L, L)] = plsc.load_gather(bufs[s], [lane + L, j])

    def batch_body(g, carry):
        pair0 = g * NBUF
        copies = [fire(pair0 + s, s) for s in range(NBUF)]
        for s in range(NBUF):
            copies[s].wait()
            extract(pair0 + s, s)
        return carry

    lax.fori_loop(0, BPW // NBUF, batch_body, 0)


@functools.partial(
    pl.kernel,
    out_type=jax.ShapeDtypeStruct((B * K,), jnp.float32),
    mesh=_mesh,
    compiler_params=_params,
    scratch_types=_gather_scratch([]),
)
def _gather_u(uidx_hbm, ut_hbm, urows_hbm, idx_v, rows, *bufs_sems):
    bufs, sems = bufs_sems[:NBUF], bufs_sems[NBUF:]
    wid = lax.axis_index("s") * NC + lax.axis_index("c")
    base = wid * BPW
    _ring_gather(uidx_hbm, ut_hbm, base, idx_v, rows, bufs, sems)
    pltpu.sync_copy(rows, urows_hbm.at[pl.ds(base * K, BPW * K)])


@functools.partial(
    pl.kernel,
    out_type=jax.ShapeDtypeStruct((B,), jnp.float32),
    mesh=_mesh,
    compiler_params=_params,
    scratch_types=_gather_scratch([
        pltpu.VMEM((BPW * K,), jnp.float32),
        pltpu.VMEM((BPW,), jnp.float32),
    ]),
)
def _gather_v_dot(vidx_hbm, vt_hbm, urows_hbm, out_hbm,
                  idx_v, rows, urows, out_v, *bufs_sems):
    bufs, sems = bufs_sems[:NBUF], bufs_sems[NBUF:]
    wid = lax.axis_index("s") * NC + lax.axis_index("c")
    base = wid * BPW

    pltpu.sync_copy(urows_hbm.at[pl.ds(base * K, BPW * K)], urows)
    _ring_gather(vidx_hbm, vt_hbm, base, idx_v, rows, bufs, sems)

    lane = lax.iota(jnp.int32, L)

    def g_body(g, carry):
        flat = (g * L + lane) * K
        acc = jnp.zeros((L,), jnp.float32)
        for k in range(K):
            acc = acc + (plsc.load_gather(urows, [flat + k]) *
                         plsc.load_gather(rows, [flat + k]))
        out_v[pl.ds(g * L, L)] = acc
        return carry

    lax.fori_loop(0, BPW // L, g_body, 0)

    pltpu.sync_copy(out_v, out_hbm.at[pl.ds(base, BPW)])


def kernel(x, user_table, item_table):
    urows = _gather_u(x[:, 0], user_table.T)
    return _gather_v_dot(x[:, 1], item_table.T, urows)


# speedup vs baseline: 3.1379x; 1.0009x over previous
import functools

import jax
import jax.numpy as jnp
from jax import lax
from jax.experimental import pallas as pl
from jax.experimental.pallas import tpu as pltpu
from jax.experimental.pallas import tpu_sc as plsc

NC = 2
NS = 16
L = 16
NW = NC * NS

B = 16384
K = 32
BPW = B // NW
NBUF = 16
W = 128

assert BPW % NBUF == 0

_mesh = plsc.VectorSubcoreMesh(
    core_axis_name="c", subcore_axis_name="s", num_cores=NC, num_subcores=NS
)

_params = pltpu.CompilerParams(needs_layout_passes=False)


def _gather_scratch(extra):
    sc = [
        pltpu.VMEM((BPW + L,), jnp.int32),
        pltpu.VMEM((BPW * K,), jnp.float32),
    ] + extra
    sc += [pltpu.VMEM((K, W), jnp.float32) for _ in range(NBUF)]
    sc += [pltpu.SemaphoreType.DMA for _ in range(NBUF)]
    return sc


def _ring_gather(idx_hbm, tab_hbm, base, idx_v, rows, bufs, sems):
    pltpu.sync_copy(idx_hbm.at[pl.ds(base, BPW)], idx_v.at[pl.ds(0, BPW)])

    lane = lax.iota(jnp.int32, L)

    def fire(pair, s):
        i0 = idx_v[pl.ds(pair, L)][0]
        t = pl.multiple_of((i0 // W) * W, W)
        return pltpu.async_copy(tab_hbm.at[:, pl.ds(t, W)], bufs[s], sems[s])

    def extract(pair, s):
        i0 = idx_v[pl.ds(pair, L)][0]
        j = jnp.full((L,), i0 % W, jnp.int32)
        rows[pl.ds(pair * K, L)] = plsc.load_gather(bufs[s], [lane, j])
        rows[pl.ds(pair * K + L, L)] = plsc.load_gather(bufs[s], [lane + L, j])

    def batch_body(g, carry):
        pair0 = g * NBUF
        copies = [fire(pair0 + s, s) for s in range(NBUF)]
        for s in range(NBUF):
            copies[s].wait()
            extract(pair0 + s, s)
        return carry

    lax.fori_loop(0, BPW // NBUF, batch_body, 0)


@functools.partial(
    pl.kernel,
    out_type=jax.ShapeDtypeStruct((B * K,), jnp.float32),
    mesh=_mesh,
    compiler_params=_params,
    scratch_types=_gather_scratch([]),
)
def _gather_u(uidx_hbm, ut_hbm, urows_hbm, idx_v, rows, *bufs_sems):
    bufs, sems = bufs_sems[:NBUF], bufs_sems[NBUF:]
    wid = lax.axis_index("s") * NC + lax.axis_index("c")
    base = wid * BPW
    _ring_gather(uidx_hbm, ut_hbm, base, idx_v, rows, bufs, sems)
    pltpu.sync_copy(rows, urows_hbm.at[pl.ds(base * K, BPW * K)])


@functools.partial(
    pl.kernel,
    out_type=jax.ShapeDtypeStruct((B,), jnp.float32),
    mesh=_mesh,
    compiler_params=_params,
    scratch_types=_gather_scratch([
        pltpu.VMEM((BPW * K,), jnp.float32),
        pltpu.VMEM((BPW,), jnp.float32),
    ]),
)
def _gather_v_dot(vidx_hbm, vt_hbm, urows_hbm, out_hbm,
                  idx_v, rows, urows, out_v, *bufs_sems):
    bufs, sems = bufs_sems[:NBUF], bufs_sems[NBUF:]
    wid = lax.axis_index("s") * NC + lax.axis_index("c")
    base = wid * BPW

    pltpu.sync_copy(urows_hbm.at[pl.ds(base * K, BPW * K)], urows)
    _ring_gather(vidx_hbm, vt_hbm, base, idx_v, rows, bufs, sems)

    lane = lax.iota(jnp.int32, L)

    def g_body(g, carry):
        flat = (g * L + lane) * K
        acc = jnp.zeros((L,), jnp.float32)
        for k in range(K):
            acc = acc + (plsc.load_gather(urows, [flat + k]) *
                         plsc.load_gather(rows, [flat + k]))
        out_v[pl.ds(g * L, L)] = acc
        return carry

    lax.fori_loop(0, BPW // L, g_body, 0)

    pltpu.sync_copy(out_v, out_hbm.at[pl.ds(base, BPW)])


def kernel(x, user_table, item_table):
    urows = _gather_u(x[:, 0], user_table.T)
    return _gather_v_dot(x[:, 1], item_table.T, urows)
